# baseline (device time: 51943 ns/iter reference)
import jax
import jax.numpy as jnp
from jax import lax
from jax.experimental import pallas as pl
from jax.experimental.pallas import tpu as pltpu

M_PER = 512
N = 256
DROWS = 4
DCOLS = 128


def kernel(x, dest):
    dest2d = dest.reshape(DROWS, DCOLS)

    def body(x_ref, d_ref, fullx_ref, fulld_ref, send_sems, recv_sems):
        my_x = lax.axis_index("x")
        my_y = lax.axis_index("y")
        my_z = lax.axis_index("z")
        other = 1 - my_x

        barrier_sem = pltpu.get_barrier_semaphore()
        pl.semaphore_signal(
            barrier_sem, inc=1,
            device_id=(other, my_y, my_z),
            device_id_type=pl.DeviceIdType.MESH,
        )
        pl.semaphore_wait(barrier_sem, 1)

        fullx_ref[pl.ds(my_x * M_PER, M_PER), :] = x_ref[:, :]
        fulld_ref[pl.ds(my_x * DROWS, DROWS), :] = d_ref[:, :]

        rdma_x = pltpu.make_async_remote_copy(
            src_ref=x_ref,
            dst_ref=fullx_ref.at[pl.ds(my_x * M_PER, M_PER), :],
            send_sem=send_sems.at[0],
            recv_sem=recv_sems.at[0],
            device_id=(other, my_y, my_z),
            device_id_type=pl.DeviceIdType.MESH,
        )
        rdma_d = pltpu.make_async_remote_copy(
            src_ref=d_ref,
            dst_ref=fulld_ref.at[pl.ds(my_x * DROWS, DROWS), :],
            send_sem=send_sems.at[1],
            recv_sem=recv_sems.at[1],
            device_id=(other, my_y, my_z),
            device_id_type=pl.DeviceIdType.MESH,
        )
        rdma_x.start()
        rdma_d.start()
        rdma_x.wait()
        rdma_d.wait()

    fullx, fulld = pl.pallas_call(
        body,
        out_shape=[
            jax.ShapeDtypeStruct((2 * M_PER, N), jnp.float32),
            jax.ShapeDtypeStruct((2 * DROWS, DCOLS), jnp.int32),
        ],
        in_specs=[
            pl.BlockSpec(memory_space=pltpu.VMEM),
            pl.BlockSpec(memory_space=pltpu.VMEM),
        ],
        out_specs=[
            pl.BlockSpec(memory_space=pltpu.VMEM),
            pl.BlockSpec(memory_space=pltpu.VMEM),
        ],
        scratch_shapes=[
            pltpu.SemaphoreType.DMA((2,)),
            pltpu.SemaphoreType.DMA((2,)),
        ],
        compiler_params=pltpu.CompilerParams(collective_id=0),
    )(x, dest2d)

    my_x = lax.axis_index("x")
    dflat = fulld.reshape(2 * M_PER)
    idx = jnp.nonzero(dflat == my_x, size=M_PER, fill_value=0)[0]
    return fullx[idx]


# device time: 15451 ns/iter; 3.3618x vs baseline; 3.3618x over previous
import jax
import jax.numpy as jnp
from jax import lax
from jax.experimental import pallas as pl
from jax.experimental.pallas import tpu as pltpu

M_PER = 512
N = 256
TOT = 2 * M_PER


def kernel(x, dest):
    dcol = dest.reshape(M_PER, 1)

    def body(x_ref, d_ref, out_ref, fullx_ref, fulld_ref, send_sems, recv_sems):
        my_x = lax.axis_index("x")
        my_y = lax.axis_index("y")
        my_z = lax.axis_index("z")
        other = 1 - my_x

        barrier_sem = pltpu.get_barrier_semaphore()
        pl.semaphore_signal(
            barrier_sem, inc=1,
            device_id=(other, my_y, my_z),
            device_id_type=pl.DeviceIdType.MESH,
        )
        pl.semaphore_wait(barrier_sem, 1)

        fullx_ref[pl.ds(my_x * M_PER, M_PER), :] = x_ref[:, :]
        fulld_ref[pl.ds(my_x * M_PER, M_PER), :] = d_ref[:, :]

        rdma_d = pltpu.make_async_remote_copy(
            src_ref=d_ref,
            dst_ref=fulld_ref.at[pl.ds(my_x * M_PER, M_PER), :],
            send_sem=send_sems.at[0],
            recv_sem=recv_sems.at[0],
            device_id=(other, my_y, my_z),
            device_id_type=pl.DeviceIdType.MESH,
        )
        rdma_x = pltpu.make_async_remote_copy(
            src_ref=x_ref,
            dst_ref=fullx_ref.at[pl.ds(my_x * M_PER, M_PER), :],
            send_sem=send_sems.at[1],
            recv_sem=recv_sems.at[1],
            device_id=(other, my_y, my_z),
            device_id_type=pl.DeviceIdType.MESH,
        )
        rdma_d.start()
        rdma_x.start()

        ia = lax.broadcasted_iota(jnp.int32, (TOT, TOT), 0)
        ja = lax.broadcasted_iota(jnp.int32, (TOT, TOT), 1)
        ltri = (ja < ia).astype(jnp.float32)
        kio = lax.broadcasted_iota(jnp.int32, (TOT, M_PER), 1).astype(jnp.float32)

        rdma_d.wait()
        match = (fulld_ref[:, :] == my_x).astype(jnp.float32)
        rank = lax.dot_general(
            ltri, match, (((1,), (0,)), ((), ())),
            preferred_element_type=jnp.float32,
        )
        pt = (rank == kio).astype(jnp.float32) * match

        rdma_x.wait()
        out_ref[:, :] = lax.dot_general(
            pt, fullx_ref[:, :], (((0,), (0,)), ((), ())),
            preferred_element_type=jnp.float32,
        )

    return pl.pallas_call(
        body,
        out_shape=jax.ShapeDtypeStruct((M_PER, N), jnp.float32),
        in_specs=[
            pl.BlockSpec(memory_space=pltpu.VMEM),
            pl.BlockSpec(memory_space=pltpu.VMEM),
        ],
        out_specs=pl.BlockSpec(memory_space=pltpu.VMEM),
        scratch_shapes=[
            pltpu.VMEM((TOT, N), jnp.float32),
            pltpu.VMEM((TOT, 1), jnp.int32),
            pltpu.SemaphoreType.DMA((2,)),
            pltpu.SemaphoreType.DMA((2,)),
        ],
        compiler_params=pltpu.CompilerParams(collective_id=0),
    )(x, dcol)


# device time: 12589 ns/iter; 4.1261x vs baseline; 1.2273x over previous
import jax
import jax.numpy as jnp
from jax import lax
from jax.experimental import pallas as pl
from jax.experimental.pallas import tpu as pltpu

M_PER = 512
N = 256
TOT = 2 * M_PER
DR = 8
DC = 64
F32 = jnp.float32


def kernel(x, dest):
    dpk = dest.reshape(DR, DC)

    def body(x_ref, d_ref, out_ref, fullx_ref, fulld_ref, send_sems, recv_sems):
        my_x = lax.axis_index("x")
        my_y = lax.axis_index("y")
        my_z = lax.axis_index("z")
        other = 1 - my_x

        barrier_sem = pltpu.get_barrier_semaphore()
        pl.semaphore_signal(
            barrier_sem, inc=1,
            device_id=(other, my_y, my_z),
            device_id_type=pl.DeviceIdType.MESH,
        )
        pl.semaphore_wait(barrier_sem, 1)

        fullx_ref[pl.ds(my_x * M_PER, M_PER), :] = x_ref[:, :]
        fulld_ref[pl.ds(my_x * DR, DR), :] = d_ref[:, :]

        rdma_d = pltpu.make_async_remote_copy(
            src_ref=d_ref,
            dst_ref=fulld_ref.at[pl.ds(my_x * DR, DR), :],
            send_sem=send_sems.at[0],
            recv_sem=recv_sems.at[0],
            device_id=(other, my_y, my_z),
            device_id_type=pl.DeviceIdType.MESH,
        )
        rdma_x = pltpu.make_async_remote_copy(
            src_ref=x_ref,
            dst_ref=fullx_ref.at[pl.ds(my_x * M_PER, M_PER), :],
            send_sem=send_sems.at[1],
            recv_sem=recv_sems.at[1],
            device_id=(other, my_y, my_z),
            device_id_type=pl.DeviceIdType.MESH,
        )
        rdma_d.start()
        rdma_x.start()

        iA = lax.broadcasted_iota(jnp.int32, (TOT, 2 * DR), 0)
        jA = lax.broadcasted_iota(jnp.int32, (TOT, 2 * DR), 1)
        selA = (lax.div(iA, DC) == jA).astype(F32)
        iC = lax.broadcasted_iota(jnp.int32, (TOT, DC), 0)
        jC = lax.broadcasted_iota(jnp.int32, (TOT, DC), 1)
        selC = (lax.rem(iC, DC) == jC).astype(F32)
        il = lax.broadcasted_iota(jnp.int32, (TOT, TOT), 0)
        jl = lax.broadcasted_iota(jnp.int32, (TOT, TOT), 1)
        ltri = (jl < il).astype(F32)
        kio = lax.broadcasted_iota(jnp.int32, (TOT, M_PER), 1).astype(F32)

        rdma_d.wait()
        mp = (fulld_ref[:, :] == my_x).astype(F32)
        rows = lax.dot_general(
            selA, mp, (((1,), (0,)), ((), ())), preferred_element_type=F32
        )
        match = jnp.sum(rows * selC, axis=1, keepdims=True)
        rank = lax.dot_general(
            ltri, match, (((1,), (0,)), ((), ())),
            preferred_element_type=F32,
        )
        pt = (rank == kio).astype(F32) * match

        rdma_x.wait()
        out_ref[:, :] = lax.dot_general(
            pt, fullx_ref[:, :], (((0,), (0,)), ((), ())),
            preferred_element_type=F32,
        )

    return pl.pallas_call(
        body,
        out_shape=jax.ShapeDtypeStruct((M_PER, N), F32),
        in_specs=[
            pl.BlockSpec(memory_space=pltpu.VMEM),
            pl.BlockSpec(memory_space=pltpu.VMEM),
        ],
        out_specs=pl.BlockSpec(memory_space=pltpu.VMEM),
        scratch_shapes=[
            pltpu.VMEM((TOT, N), F32),
            pltpu.VMEM((2 * DR, DC), jnp.int32),
            pltpu.SemaphoreType.DMA((2,)),
            pltpu.SemaphoreType.DMA((2,)),
        ],
        compiler_params=pltpu.CompilerParams(collective_id=0),
    )(x, dpk)


# device time: 10833 ns/iter; 4.7949x vs baseline; 1.1621x over previous
import jax
import jax.numpy as jnp
from jax import lax
from jax.experimental import pallas as pl
from jax.experimental.pallas import tpu as pltpu

M_PER = 512
N = 256
DR = 8
DC = 64
CH = 64
NCH = M_PER // CH
F32 = jnp.float32


def kernel(x, dest):
    dpk = dest.reshape(DR, DC)

    def body(x_ref, d_ref, out_ref, send_ref, recv_ref, send_sems, recv_sems):
        my_x = lax.axis_index("x")
        my_y = lax.axis_index("y")
        my_z = lax.axis_index("z")
        other = 1 - my_x

        barrier_sem = pltpu.get_barrier_semaphore()
        pl.semaphore_signal(
            barrier_sem, inc=1,
            device_id=(other, my_y, my_z),
            device_id_type=pl.DeviceIdType.MESH,
        )
        pl.semaphore_wait(barrier_sem, 1)

        iA = lax.broadcasted_iota(jnp.int32, (M_PER, DR), 0)
        jA = lax.broadcasted_iota(jnp.int32, (M_PER, DR), 1)
        selA = (lax.div(iA, DC) == jA).astype(F32)
        iC = lax.broadcasted_iota(jnp.int32, (M_PER, DC), 0)
        jC = lax.broadcasted_iota(jnp.int32, (M_PER, DC), 1)
        selC = (lax.rem(iC, DC) == jC).astype(F32)
        il = lax.broadcasted_iota(jnp.int32, (M_PER, M_PER), 0)
        jl = lax.broadcasted_iota(jnp.int32, (M_PER, M_PER), 1)
        ltri = (jl < il).astype(F32)
        kio = jl.astype(F32)
        icol = lax.broadcasted_iota(jnp.int32, (M_PER, 1), 0).astype(F32)

        mp_send = (d_ref[:, :] == other).astype(F32)
        rows = lax.dot_general(
            selA, mp_send, (((1,), (0,)), ((), ())), preferred_element_type=F32
        )
        match_s = jnp.sum(rows * selC, axis=1, keepdims=True)
        rank_s = lax.dot_general(
            ltri, match_s, (((1,), (0,)), ((), ())), preferred_element_type=F32
        )
        n = jnp.sum(match_s).astype(jnp.int32)
        nf = n.astype(F32)

        st = (rank_s == kio).astype(F32) * match_s
        send_ref[:, :] = lax.dot_general(
            st, x_ref[:, :], (((0,), (0,)), ((), ())), preferred_element_type=F32
        )

        rdmas = []
        for j in range(NCH):
            rdma = pltpu.make_async_remote_copy(
                src_ref=send_ref.at[pl.ds(j * CH, CH), :],
                dst_ref=recv_ref.at[pl.ds(j * CH, CH), :],
                send_sem=send_sems.at[j],
                recv_sem=recv_sems.at[j],
                device_id=(other, my_y, my_z),
                device_id_type=pl.DeviceIdType.MESH,
            )
            rdmas.append(rdma)

            @pl.when(j * CH < n)
            def _(rdma=rdma):
                rdma.start()

        match_k = 1.0 - match_s
        rank_k = icol - rank_s
        off_k = jnp.where(my_x == 1, nf, 0.0)
        ptk = (rank_k + off_k == kio).astype(F32) * match_k
        keep_part = lax.dot_general(
            ptk, x_ref[:, :], (((0,), (0,)), ((), ())), preferred_element_type=F32
        )

        off_r = jnp.where(my_x == 0, 512.0 - nf, 0.0)
        ir = lax.broadcasted_iota(jnp.int32, (M_PER, M_PER), 0).astype(F32)
        ptr = ((ir + off_r == kio) & (ir < nf)).astype(F32)

        for j in range(NCH):
            @pl.when(j * CH < n)
            def _(rdma=rdmas[j]):
                rdma.wait_send()
                rdma.wait_recv()

        out_ref[:, :] = keep_part + lax.dot_general(
            ptr, recv_ref[:, :], (((0,), (0,)), ((), ())),
            preferred_element_type=F32,
        )

    return pl.pallas_call(
        body,
        out_shape=jax.ShapeDtypeStruct((M_PER, N), F32),
        in_specs=[
            pl.BlockSpec(memory_space=pltpu.VMEM),
            pl.BlockSpec(memory_space=pltpu.VMEM),
        ],
        out_specs=pl.BlockSpec(memory_space=pltpu.VMEM),
        scratch_shapes=[
            pltpu.VMEM((M_PER, N), F32),
            pltpu.VMEM((M_PER, N), F32),
            pltpu.SemaphoreType.DMA((NCH,)),
            pltpu.SemaphoreType.DMA((NCH,)),
        ],
        compiler_params=pltpu.CompilerParams(collective_id=0),
    )(x, dpk)


# device time: 10441 ns/iter; 4.9749x vs baseline; 1.0375x over previous
import jax
import jax.numpy as jnp
from jax import lax
from jax.experimental import pallas as pl
from jax.experimental.pallas import tpu as pltpu

M_PER = 512
N = 256
DR = 8
DC = 64
CH = 64
NCH = M_PER // CH
F32 = jnp.float32


def kernel(x, dest):
    dpk = dest.reshape(DR, DC)

    def body(x_ref, d_ref, out_ref, send_ref, recv_ref, send_sems, recv_sems):
        my_x = lax.axis_index("x")
        my_y = lax.axis_index("y")
        my_z = lax.axis_index("z")
        other = 1 - my_x

        barrier_sem = pltpu.get_barrier_semaphore()
        pl.semaphore_signal(
            barrier_sem, inc=1,
            device_id=(other, my_y, my_z),
            device_id_type=pl.DeviceIdType.MESH,
        )

        iA = lax.broadcasted_iota(jnp.int32, (M_PER, DR), 0)
        jA = lax.broadcasted_iota(jnp.int32, (M_PER, DR), 1)
        selA = (lax.div(iA, DC) == jA).astype(F32)
        iC = lax.broadcasted_iota(jnp.int32, (M_PER, DC), 0)
        jC = lax.broadcasted_iota(jnp.int32, (M_PER, DC), 1)
        selC = (lax.rem(iC, DC) == jC).astype(F32)
        il = lax.broadcasted_iota(jnp.int32, (M_PER, M_PER), 0)
        jl = lax.broadcasted_iota(jnp.int32, (M_PER, M_PER), 1)
        ltri = (jl < il).astype(F32)
        kio = jl.astype(F32)
        icol = lax.broadcasted_iota(jnp.int32, (M_PER, 1), 0).astype(F32)

        mp_send = (d_ref[:, :] == other).astype(F32)
        rows = lax.dot_general(
            selA, mp_send, (((1,), (0,)), ((), ())), preferred_element_type=F32
        )
        match_s = jnp.sum(rows * selC, axis=1, keepdims=True)
        rank_s = lax.dot_general(
            ltri, match_s, (((1,), (0,)), ((), ())), preferred_element_type=F32
        )
        n = jnp.sum(match_s).astype(jnp.int32)
        nf = n.astype(F32)

        st = (rank_s == kio).astype(F32) * match_s

        pl.semaphore_wait(barrier_sem, 1)

        rdmas = []
        for j in range(NCH):
            rdma = pltpu.make_async_remote_copy(
                src_ref=send_ref.at[pl.ds(j * CH, CH), :],
                dst_ref=recv_ref.at[pl.ds(j * CH, CH), :],
                send_sem=send_sems.at[j],
                recv_sem=recv_sems.at[j],
                device_id=(other, my_y, my_z),
                device_id_type=pl.DeviceIdType.MESH,
            )
            rdmas.append(rdma)

            @pl.when(j * CH < n)
            def _(rdma=rdma, j=j):
                send_ref[pl.ds(j * CH, CH), :] = lax.dot_general(
                    st[:, j * CH:(j + 1) * CH], x_ref[:, :],
                    (((0,), (0,)), ((), ())), preferred_element_type=F32,
                )
                rdma.start()

        match_k = 1.0 - match_s
        rank_k = icol - rank_s
        off_k = jnp.where(my_x == 1, nf, 0.0)
        ptk = (rank_k + off_k == kio).astype(F32) * match_k
        keep_part = lax.dot_general(
            ptk, x_ref[:, :], (((0,), (0,)), ((), ())), preferred_element_type=F32
        )

        off_r = jnp.where(my_x == 0, 512.0 - nf, 0.0)
        ir = lax.broadcasted_iota(jnp.int32, (M_PER, M_PER), 0).astype(F32)
        ptr = ((ir + off_r == kio) & (ir < nf)).astype(F32)

        for j in range(NCH):
            @pl.when(j * CH < n)
            def _(rdma=rdmas[j]):
                rdma.wait_send()
                rdma.wait_recv()

        out_ref[:, :] = keep_part + lax.dot_general(
            ptr, recv_ref[:, :], (((0,), (0,)), ((), ())),
            preferred_element_type=F32,
        )

    return pl.pallas_call(
        body,
        out_shape=jax.ShapeDtypeStruct((M_PER, N), F32),
        in_specs=[
            pl.BlockSpec(memory_space=pltpu.VMEM),
            pl.BlockSpec(memory_space=pltpu.VMEM),
        ],
        out_specs=pl.BlockSpec(memory_space=pltpu.VMEM),
        scratch_shapes=[
            pltpu.VMEM((M_PER, N), F32),
            pltpu.VMEM((M_PER, N), F32),
            pltpu.SemaphoreType.DMA((NCH,)),
            pltpu.SemaphoreType.DMA((NCH,)),
        ],
        compiler_params=pltpu.CompilerParams(collective_id=0),
    )(x, dpk)
